# baseline (speedup 1.0000x reference)
import functools

import jax
import jax.numpy as jnp
from jax import lax
from jax.experimental import pallas as pl
from jax.experimental.pallas import tpu as pltpu
from jax.experimental.pallas import tpu_sc as plsc

D0, D1, D2, C = 128, 160, 192, 2
N = D0 * D1 * D2

NC, NS = 2, 16
NW = NC * NS
X0_PER_W = D0 // NW
ROWS_PER_BLOCK = 2
B = ROWS_PER_BLOCK * D2
NIDX = B // 128
CHUNKS_PER_ROW = D2 // 16
BLOCKS_PER_X0 = D1 // ROWS_PER_BLOCK

_MESH = plsc.VectorSubcoreMesh(core_axis_name="c", subcore_axis_name="s")


def _body(tab_ref, df_ref, out_ref, df_v, idx_v, w_v, gat_v, par_v, out_v, sem):
  wid = lax.axis_index("s") * NC + lax.axis_index("c")
  lane = lax.iota(jnp.int32, 16)
  lane3 = lane * 3
  lane_f = lane.astype(jnp.float32)

  def splat(x):
    return jnp.full((16,), x, jnp.int32)

  def corner1d(loc, maxd):
    lc = jnp.clip(loc, -4.0, float(maxd) + 4.0)
    t0 = lc.astype(jnp.int32)
    fl = jnp.where(t0.astype(jnp.float32) > lc, t0 - 1, t0)
    a0 = jnp.clip(fl, 0, maxd)
    a1 = jnp.minimum(a0 + 1, maxd)
    w0 = a1.astype(jnp.float32) - loc
    return a0, a1, w0

  def run_block(x0, bb):
    vbase = x0 * (D1 * D2) + bb * B

    pltpu.sync_copy(df_ref.at[pl.ds(3 * vbase, 3 * B)], df_v)

    for j in range(ROWS_PER_BLOCK):
      for tz in range(CHUNKS_PER_ROW):
        x1 = bb * ROWS_PER_BLOCK + j
        voff = j * D2 + tz * 16
        r3 = 3 * voff + lane3
        dfx = plsc.load_gather(df_v, [r3])
        dfy = plsc.load_gather(df_v, [r3 + 1])
        dfz = plsc.load_gather(df_v, [r3 + 2])
        locx = x0.astype(jnp.float32) + dfx
        locy = x1.astype(jnp.float32) + dfy
        locz = float(tz * 16) + lane_f + dfz

        ax0, ax1, wx0 = corner1d(locx, D0 - 1)
        ay0, ay1, wy0 = corner1d(locy, D1 - 1)
        az0, az1, wz0 = corner1d(locz, D2 - 1)
        wx1 = 1.0 - wx0
        wy1 = 1.0 - wy0
        wz1 = 1.0 - wz0
        zedge = az1 == az0
        wz0 = jnp.where(zedge, 1.0, wz0)
        wz1 = jnp.where(zedge, 0.0, wz1)

        k = voff // 128
        o = voff % 128
        for ci, (ax, ay) in enumerate(
            ((ax0, ay0), (ax0, ay1), (ax1, ay0), (ax1, ay1))):
          q = (ax * D1 + ay) * D2 + az0
          idx_v[ci * NIDX + k, pl.ds(o, 16)] = lax.shift_right_logical(q, 1)
        par_v[pl.ds(voff, 16)] = (az0 & 1) * 2
        w_v[0, pl.ds(voff, 16)] = wx0 * wy0
        w_v[1, pl.ds(voff, 16)] = wx0 * wy1
        w_v[2, pl.ds(voff, 16)] = wx1 * wy0
        w_v[3, pl.ds(voff, 16)] = wx1 * wy1
        w_v[4, pl.ds(voff, 16)] = wz0
        w_v[5, pl.ds(voff, 16)] = wz1

    cps = []
    for m in range(4 * NIDX):
      cps.append(pltpu.async_copy(
          tab_ref.at[idx_v.at[m]], gat_v.at[m], sem))
    for cp in cps:
      cp.wait()

    for t in range(ROWS_PER_BLOCK * CHUNKS_PER_ROW):
      voff = t * 16
      k = voff // 128
      o = voff % 128
      rows = o + lane
      p2 = par_v[pl.ds(voff, 16)]
      wz0 = w_v[4, pl.ds(voff, 16)]
      wz1 = w_v[5, pl.ds(voff, 16)]
      acc0 = jnp.zeros((16,), jnp.float32)
      acc1 = jnp.zeros((16,), jnp.float32)
      for ci in range(4):
        wxy = w_v[ci, pl.ds(voff, 16)]
        m = splat(ci * NIDX + k)
        gz0c0 = plsc.load_gather(gat_v, [m, rows, p2])
        gz0c1 = plsc.load_gather(gat_v, [m, rows, p2 + 1])
        gz1c0 = plsc.load_gather(gat_v, [m, rows, p2 + 2])
        gz1c1 = plsc.load_gather(gat_v, [m, rows, p2 + 3])
        acc0 = acc0 + wxy * (wz0 * gz0c0 + wz1 * gz1c0)
        acc1 = acc1 + wxy * (wz0 * gz0c1 + wz1 * gz1c1)
      oidx = 2 * (voff + lane)
      plsc.store_scatter(out_v, [oidx], acc0)
      plsc.store_scatter(out_v, [oidx + 1], acc1)

    pltpu.sync_copy(out_v, out_ref.at[pl.ds(2 * vbase, 2 * B)])

  def x0_body(a, _):
    x0 = wid * X0_PER_W + a

    def bb_body(bb, _):
      run_block(x0, bb)
      return 0

    lax.fori_loop(0, BLOCKS_PER_X0, bb_body, 0)
    return 0

  lax.fori_loop(0, X0_PER_W, x0_body, 0)


@jax.jit
def _interp(tab1d, dff):
  tab = tab1d.reshape(N // 2, 8)
  kfn = pl.kernel(
      _body,
      out_type=jax.ShapeDtypeStruct((2 * N,), jnp.float32),
      mesh=_MESH,
      compiler_params=pltpu.CompilerParams(
          needs_layout_passes=False, use_tc_tiling_on_sc=False),
      scratch_types=[
          pltpu.VMEM((3 * B,), jnp.float32),
          pltpu.VMEM((4 * NIDX, 128), jnp.int32),
          pltpu.VMEM((6, B), jnp.float32),
          pltpu.VMEM((4 * NIDX, 128, 8), jnp.float32),
          pltpu.VMEM((B,), jnp.int32),
          pltpu.VMEM((2 * B,), jnp.float32),
          pltpu.SemaphoreType.DMA,
      ],
  )
  return kfn(tab, dff)


def kernel(vol, df):
  vf = vol.reshape(-1)
  pt = jnp.concatenate([vf, jnp.zeros((4,), vf.dtype)])
  tab1d = jnp.concatenate(
      [pt[:2 * N].reshape(N // 2, 4), pt[4:].reshape(N // 2, 4)],
      axis=1).reshape(4 * N)
  out = _interp(tab1d, df.reshape(3 * N))
  return out.reshape(D0, D1, D2, C)


# speedup vs baseline: 2.5877x; 2.5877x over previous
import jax
import jax.numpy as jnp
from jax import lax
from jax.experimental import pallas as pl
from jax.experimental.pallas import tpu as pltpu
from jax.experimental.pallas import tpu_sc as plsc

D0, D1, D2, C = 128, 160, 192, 2
N = D0 * D1 * D2

NC, NS = 2, 16
NW = NC * NS
X1_PER_W = D1 // NW
ZB = 3
B = ZB * D0
NIDX = B // 128
XCHUNKS = D0 // 16
BLOCKS_PER_X1 = D2 // ZB

_MESH = plsc.VectorSubcoreMesh(core_axis_name="c", subcore_axis_name="s")


def _body(tab_ref, df_ref, out_ref, df_v, idx_v, w_v, gat_v, par_v, out_v, sem):
  wid = lax.axis_index("s") * NC + lax.axis_index("c")
  lane = lax.iota(jnp.int32, 16)
  lane_f = lane.astype(jnp.float32)

  def splat(x):
    return jnp.full((16,), x, jnp.int32)

  def corner1d(loc, maxd):
    lc = jnp.clip(loc, -4.0, float(maxd) + 4.0)
    t0 = lc.astype(jnp.int32)
    fl = jnp.where(t0.astype(jnp.float32) > lc, t0 - 1, t0)
    a0 = jnp.clip(fl, 0, maxd)
    a1 = jnp.minimum(a0 + 1, maxd)
    w0 = a1.astype(jnp.float32) - loc
    return a0, a1, w0

  def run_block(x1, zb):
    z0 = zb * ZB
    for d in range(3):
      pltpu.sync_copy(df_ref.at[x1, d, pl.ds(z0, ZB)],
                      df_v.at[pl.ds(d * ZB, ZB)])

    for zl in range(ZB):
      for xc in range(XCHUNKS):
        voff = zl * D0 + xc * 16
        xsl = pl.ds(xc * 16, 16)
        dfx = df_v[0 * ZB + zl, xsl]
        dfy = df_v[1 * ZB + zl, xsl]
        dfz = df_v[2 * ZB + zl, xsl]
        locx = float(xc * 16) + lane_f + dfx
        locy = x1.astype(jnp.float32) + dfy
        locz = (z0 + zl).astype(jnp.float32) + dfz

        ax0, ax1, wx0 = corner1d(locx, D0 - 1)
        ay0, ay1, wy0 = corner1d(locy, D1 - 1)
        az0, az1, wz0 = corner1d(locz, D2 - 1)
        wy1 = 1.0 - wy0
        wz1 = 1.0 - wz0
        wx1 = 1.0 - wx0
        xedge = ax1 == ax0
        wx0 = jnp.where(xedge, 1.0, wx0)
        wx1 = jnp.where(xedge, 0.0, wx1)

        k = voff // 128
        o = voff % 128
        xo = lax.shift_right_logical(ax0, 2)
        for ci, (ay, az) in enumerate(
            ((ay0, az0), (ay0, az1), (ay1, az0), (ay1, az1))):
          base = (ay * (D2 * 2) + az * 2) * 32 + xo
          idx_v[(2 * ci) * NIDX + k, pl.ds(o, 16)] = base
          idx_v[(2 * ci + 1) * NIDX + k, pl.ds(o, 16)] = base + 32
        par_v[pl.ds(voff, 16)] = ax0 & 3
        w_v[0, pl.ds(voff, 16)] = wy0 * wz0
        w_v[1, pl.ds(voff, 16)] = wy0 * wz1
        w_v[2, pl.ds(voff, 16)] = wy1 * wz0
        w_v[3, pl.ds(voff, 16)] = wy1 * wz1
        w_v[4, pl.ds(voff, 16)] = wx0
        w_v[5, pl.ds(voff, 16)] = wx1

    cps = []
    for m in range(8 * NIDX):
      cps.append(pltpu.async_copy(
          tab_ref.at[idx_v.at[m]], gat_v.at[m], sem))
    for cp in cps:
      cp.wait()

    for t in range(ZB * XCHUNKS):
      voff = t * 16
      k = voff // 128
      o = voff % 128
      rows = o + lane
      par = par_v[pl.ds(voff, 16)]
      wx0 = w_v[4, pl.ds(voff, 16)]
      wx1 = w_v[5, pl.ds(voff, 16)]
      acc0 = jnp.zeros((16,), jnp.float32)
      acc1 = jnp.zeros((16,), jnp.float32)
      for ci in range(4):
        wyz = w_v[ci, pl.ds(voff, 16)]
        m0 = splat((2 * ci) * NIDX + k)
        m1 = splat((2 * ci + 1) * NIDX + k)
        g0a = plsc.load_gather(gat_v, [m0, rows, par])
        g0b = plsc.load_gather(gat_v, [m0, rows, par + 1])
        g1a = plsc.load_gather(gat_v, [m1, rows, par])
        g1b = plsc.load_gather(gat_v, [m1, rows, par + 1])
        acc0 = acc0 + wyz * (wx0 * g0a + wx1 * g0b)
        acc1 = acc1 + wyz * (wx0 * g1a + wx1 * g1b)
      oidx = 2 * (voff + lane)
      plsc.store_scatter(out_v, [oidx], acc0)
      plsc.store_scatter(out_v, [oidx + 1], acc1)

    vbase = (x1 * D2 + z0) * D0
    pltpu.sync_copy(out_v, out_ref.at[pl.ds(2 * vbase, 2 * B)])

  def x1_body(a, _):
    x1 = wid * X1_PER_W + a

    def zb_body(zb, _):
      run_block(x1, zb)
      return 0

    lax.fori_loop(0, BLOCKS_PER_X1, zb_body, 0)
    return 0

  lax.fori_loop(0, X1_PER_W, x1_body, 0)


@jax.jit
def _interp(tab1d, dfT):
  tab = tab1d.reshape(N // 2, 8)
  kfn = pl.kernel(
      _body,
      out_type=jax.ShapeDtypeStruct((2 * N,), jnp.float32),
      mesh=_MESH,
      compiler_params=pltpu.CompilerParams(
          needs_layout_passes=False, use_tc_tiling_on_sc=False),
      scratch_types=[
          pltpu.VMEM((3 * ZB, 128), jnp.float32),
          pltpu.VMEM((8 * NIDX, 128), jnp.int32),
          pltpu.VMEM((6, B), jnp.float32),
          pltpu.VMEM((8 * NIDX, 128, 8), jnp.float32),
          pltpu.VMEM((B,), jnp.int32),
          pltpu.VMEM((2 * B,), jnp.float32),
          pltpu.SemaphoreType.DMA,
      ],
  )
  return kfn(tab, dfT)


def kernel(vol, df):
  volp = jnp.transpose(vol, (1, 2, 3, 0)).reshape(2 * N)
  dfT = jnp.transpose(df, (1, 3, 2, 0))
  pt = jnp.concatenate([volp, jnp.zeros((4,), volp.dtype)])
  tab1d = jnp.concatenate(
      [pt[:2 * N].reshape(N // 2, 4), pt[4:].reshape(N // 2, 4)],
      axis=1).reshape(4 * N)
  out = _interp(tab1d, dfT)
  return out.reshape(D1, D2, D0, C).transpose(2, 0, 1, 3)


# speedup vs baseline: 4.0347x; 1.5592x over previous
import jax
import jax.numpy as jnp
from jax import lax
from jax.experimental import pallas as pl
from jax.experimental.pallas import tpu as pltpu
from jax.experimental.pallas import tpu_sc as plsc

D0, D1, D2, C = 128, 160, 192, 2
N = D0 * D1 * D2

NC, NS = 2, 16
NW = NC * NS
X1_PER_W = D1 // NW
ZB = 3
B = ZB * D0
NIDX = B // 128
XCHUNKS = D0 // 16
BLOCKS_PER_X1 = D2 // ZB

_MESH = plsc.VectorSubcoreMesh(core_axis_name="c", subcore_axis_name="s")


def _body(tab_ref, df_ref, out_ref, df_v, idx_v, w_v, gat_v, par_v, out_v, sem):
  wid = lax.axis_index("s") * NC + lax.axis_index("c")
  lane = lax.iota(jnp.int32, 16)
  lane_f = lane.astype(jnp.float32)

  def splat(x):
    return jnp.full((16,), x, jnp.int32)

  def corner1d(loc, maxd):
    lc = jnp.clip(loc, -4.0, float(maxd) + 4.0)
    t0 = lc.astype(jnp.int32)
    fl = jnp.where(t0.astype(jnp.float32) > lc, t0 - 1, t0)
    a0 = jnp.clip(fl, 0, maxd)
    a1 = jnp.minimum(a0 + 1, maxd)
    w0 = a1.astype(jnp.float32) - loc
    return a0, a1, w0

  def run_block(x1, zb):
    z0 = zb * ZB
    for d in range(3):
      pltpu.sync_copy(df_ref.at[x1, d, pl.ds(z0, ZB)],
                      df_v.at[pl.ds(d * ZB, ZB)])

    for zl in range(ZB):
      for xc in range(XCHUNKS):
        voff = zl * D0 + xc * 16
        xsl = pl.ds(xc * 16, 16)
        dfx = df_v[0 * ZB + zl, xsl]
        dfy = df_v[1 * ZB + zl, xsl]
        dfz = df_v[2 * ZB + zl, xsl]
        locx = float(xc * 16) + lane_f + dfx
        locy = x1.astype(jnp.float32) + dfy
        locz = (z0 + zl).astype(jnp.float32) + dfz

        ax0, ax1, wx0 = corner1d(locx, D0 - 1)
        ay0, ay1, wy0 = corner1d(locy, D1 - 1)
        az0, az1, wz0 = corner1d(locz, D2 - 1)
        wy1 = 1.0 - wy0
        wz1 = 1.0 - wz0
        wx1 = 1.0 - wx0
        xedge = ax1 == ax0
        wx0 = jnp.where(xedge, 1.0, wx0)
        wx1 = jnp.where(xedge, 0.0, wx1)

        k = voff // 128
        o = voff % 128
        a7 = ax0 & 7
        sel = a7 == 7
        xq = jnp.where(sel, lax.shift_right_logical(ax0 - 4, 3) + (N // 4),
                       lax.shift_right_logical(ax0, 3))
        for ci, (ay, az) in enumerate(
            ((ay0, az0), (ay0, az1), (ay1, az0), (ay1, az1))):
          base = (ay * (D2 * 2) + az * 2) * 16 + xq
          idx_v[(2 * ci) * NIDX + k, pl.ds(o, 16)] = base
          idx_v[(2 * ci + 1) * NIDX + k, pl.ds(o, 16)] = base + 16
        par_v[pl.ds(voff, 16)] = jnp.where(sel, 3, a7)
        w_v[0, pl.ds(voff, 16)] = wy0 * wz0
        w_v[1, pl.ds(voff, 16)] = wy0 * wz1
        w_v[2, pl.ds(voff, 16)] = wy1 * wz0
        w_v[3, pl.ds(voff, 16)] = wy1 * wz1
        w_v[4, pl.ds(voff, 16)] = wx0
        w_v[5, pl.ds(voff, 16)] = wx1

    cps = []
    for m in range(8 * NIDX):
      cps.append(pltpu.async_copy(
          tab_ref.at[idx_v.at[m]], gat_v.at[m], sem))
    for cp in cps:
      cp.wait()

    for t in range(ZB * XCHUNKS):
      voff = t * 16
      k = voff // 128
      o = voff % 128
      rows = o + lane
      par = par_v[pl.ds(voff, 16)]
      wx0 = w_v[4, pl.ds(voff, 16)]
      wx1 = w_v[5, pl.ds(voff, 16)]
      acc0 = jnp.zeros((16,), jnp.float32)
      acc1 = jnp.zeros((16,), jnp.float32)
      for ci in range(4):
        wyz = w_v[ci, pl.ds(voff, 16)]
        m0 = splat((2 * ci) * NIDX + k)
        m1 = splat((2 * ci + 1) * NIDX + k)
        g0a = plsc.load_gather(gat_v, [m0, rows, par])
        g0b = plsc.load_gather(gat_v, [m0, rows, par + 1])
        g1a = plsc.load_gather(gat_v, [m1, rows, par])
        g1b = plsc.load_gather(gat_v, [m1, rows, par + 1])
        acc0 = acc0 + wyz * (wx0 * g0a + wx1 * g0b)
        acc1 = acc1 + wyz * (wx0 * g1a + wx1 * g1b)
      oidx = 2 * (voff + lane)
      plsc.store_scatter(out_v, [oidx], acc0)
      plsc.store_scatter(out_v, [oidx + 1], acc1)

    vbase = (x1 * D2 + z0) * D0
    pltpu.sync_copy(out_v, out_ref.at[pl.ds(2 * vbase, 2 * B)])

  def x1_body(a, _):
    x1 = wid * X1_PER_W + a

    def zb_body(zb, _):
      run_block(x1, zb)
      return 0

    lax.fori_loop(0, BLOCKS_PER_X1, zb_body, 0)
    return 0

  lax.fori_loop(0, X1_PER_W, x1_body, 0)


@jax.jit
def _interp(tab1d, dfT):
  tab = tab1d.reshape(N // 2, 8)
  kfn = pl.kernel(
      _body,
      out_type=jax.ShapeDtypeStruct((2 * N,), jnp.float32),
      mesh=_MESH,
      compiler_params=pltpu.CompilerParams(
          needs_layout_passes=False, use_tc_tiling_on_sc=False),
      scratch_types=[
          pltpu.VMEM((3 * ZB, 128), jnp.float32),
          pltpu.VMEM((8 * NIDX, 128), jnp.int32),
          pltpu.VMEM((6, B), jnp.float32),
          pltpu.VMEM((8 * NIDX, 128, 8), jnp.float32),
          pltpu.VMEM((B,), jnp.int32),
          pltpu.VMEM((2 * B,), jnp.float32),
          pltpu.SemaphoreType.DMA,
      ],
  )
  return kfn(tab, dfT)


def kernel(vol, df):
  volp = jnp.transpose(vol, (1, 2, 3, 0)).reshape(2 * N)
  dfT = jnp.transpose(df, (1, 3, 2, 0))
  tab1d = jnp.concatenate(
      [volp, volp[4:], jnp.zeros((4,), volp.dtype)])
  out = _interp(tab1d, dfT)
  return out.reshape(D1, D2, D0, C).transpose(2, 0, 1, 3)
